# Initial kernel scaffold; baseline (speedup 1.0000x reference)
import functools

import jax
import jax.numpy as jnp
from jax import lax
from jax.experimental import pallas as pl
from jax.experimental.pallas import tpu as pltpu
from jax.experimental.pallas import tpu_sc as plsc

N_USERS = 4000
N_NODES = 10000
N_EDGES = 320000
D = 128
N_LAYERS = 3

NC = 2
NS = 16
BLK = 128
JB = 80
NE_PAD = NC * NS * JB * BLK
NACC = 10240
ZROWS = 128

_mesh = plsc.VectorSubcoreMesh(
    core_axis_name="c", subcore_axis_name="s", num_cores=NC, num_subcores=NS)


@functools.partial(
    pl.kernel,
    out_type=jax.ShapeDtypeStruct((NC, NACC, D), jnp.float32),
    mesh=_mesh,
    scratch_types=[
        pltpu.VMEM((JB, BLK), jnp.int32),
        pltpu.VMEM((JB, BLK), jnp.int32),
        pltpu.VMEM((JB, BLK), jnp.float32),
        pltpu.VMEM((BLK, D), jnp.float32),
        pltpu.VMEM((ZROWS, D), jnp.float32),
        pltpu.VMEM_SHARED((NACC, D), jnp.float32),
    ],
)
def _sc_propagate(x_hbm, col_hbm, row_hbm, val_hbm, part_hbm,
                  col_v, row_v, val_v, gbuf, zbuf, acc):
    c = lax.axis_index("c")
    s = lax.axis_index("s")

    pltpu.sync_copy(col_hbm.at[c, s], col_v)
    pltpu.sync_copy(row_hbm.at[c, s], row_v)
    pltpu.sync_copy(val_hbm.at[c, s], val_v)

    def _zrow(i, carry):
        for cc in range(D // 16):
            zbuf[i, pl.ds(cc * 16, 16)] = jnp.zeros((16,), jnp.float32)
        return carry
    lax.fori_loop(0, ZROWS, _zrow, 0)
    my0 = s * (NACC // NS)
    for k in range(NACC // NS // ZROWS):
        pltpu.sync_copy(zbuf, acc.at[pl.ds(my0 + k * ZROWS, ZROWS)])

    plsc.subcore_barrier()

    def _block(j, carry):
        pltpu.sync_copy(x_hbm.at[col_v.at[j]], gbuf)

        def _srow(r, carry2):
            jv = jnp.full((16,), j, jnp.int32)
            rv = jnp.full((16,), r, jnp.int32)
            bb = plsc.load_gather(val_v, (jv, rv))
            for cc in range(D // 16):
                sl = pl.ds(cc * 16, 16)
                gbuf[r, sl] = gbuf[r, sl] * bb
            return carry2
        lax.fori_loop(0, BLK, _srow, 0)

        pltpu.sync_copy(gbuf, acc.at[row_v.at[j]], add=True)
        return carry
    lax.fori_loop(0, JB, _block, 0)

    plsc.subcore_barrier()

    for k in range(NACC // NS // ZROWS):
        off = my0 + k * ZROWS
        pltpu.sync_copy(acc.at[pl.ds(off, ZROWS)], zbuf)
        pltpu.sync_copy(zbuf, part_hbm.at[c].at[pl.ds(off, ZROWS)])


def _combine_body(last, part_ref, mean_ref, x_ref, mout_ref):
    x = part_ref[0] + part_ref[1]
    x_ref[...] = x
    m = mean_ref[...] + x
    mout_ref[...] = m * 0.25 if last else m


def _tc_combine(part, mean, last):
    r = 1000
    grid = N_NODES // r
    return pl.pallas_call(
        functools.partial(_combine_body, last),
        grid=(grid,),
        in_specs=[
            pl.BlockSpec((NC, r, D), lambda i: (0, i, 0)),
            pl.BlockSpec((r, D), lambda i: (i, 0)),
        ],
        out_specs=[
            pl.BlockSpec((r, D), lambda i: (i, 0)),
            pl.BlockSpec((r, D), lambda i: (i, 0)),
        ],
        out_shape=[
            jax.ShapeDtypeStruct((N_NODES, D), jnp.float32),
            jax.ShapeDtypeStruct((N_NODES, D), jnp.float32),
        ],
    )(part, mean)


def kernel(emb, adj_val, adj_row, adj_col):
    col = adj_col.astype(jnp.int32)
    row = adj_row.astype(jnp.int32)
    val = adj_val.astype(jnp.float32)
    pad = NE_PAD - N_EDGES
    col = jnp.concatenate([col, jnp.zeros((pad,), jnp.int32)])
    row = jnp.concatenate([row, jnp.full((pad,), NACC - 1, jnp.int32)])
    val = jnp.concatenate([val, jnp.zeros((pad,), jnp.float32)])
    col3 = col.reshape(NC, NS, JB, BLK)
    row3 = row.reshape(NC, NS, JB, BLK)
    val3 = val.reshape(NC, NS, JB, BLK)

    x = emb
    mean = emb
    for layer in range(N_LAYERS):
        part = _sc_propagate(x, col3, row3, val3)
        x, mean = _tc_combine(part, mean, layer == N_LAYERS - 1)
    return mean[:N_USERS], mean[N_USERS:]


# speedup vs baseline: 2.3671x; 2.3671x over previous
import functools

import jax
import jax.numpy as jnp
from jax import lax
from jax.experimental import pallas as pl
from jax.experimental.pallas import tpu as pltpu
from jax.experimental.pallas import tpu_sc as plsc

N_USERS = 4000
N_NODES = 10000
N_EDGES = 320000
D = 128
N_LAYERS = 3

NC = 2
NS = 16
BLK = 128
JB = 80
NE_PAD = NC * NS * JB * BLK
NACC = 10240
ZROWS = 128

_mesh = plsc.VectorSubcoreMesh(
    core_axis_name="c", subcore_axis_name="s", num_cores=NC, num_subcores=NS)


@functools.partial(
    pl.kernel,
    out_type=jax.ShapeDtypeStruct((NC, NACC, D), jnp.float32),
    mesh=_mesh,
    scratch_types=[
        pltpu.VMEM((JB, BLK), jnp.int32),
        pltpu.VMEM((JB, BLK), jnp.int32),
        pltpu.VMEM((JB, BLK), jnp.float32),
        pltpu.VMEM((BLK, D), jnp.float32),
        pltpu.VMEM_SHARED((NACC, D), jnp.float32),
    ],
)
def _sc_propagate(x_hbm, col_hbm, row_hbm, val_hbm, part_hbm,
                  col_v, row_v, val_v, gbuf, acc):
    c = lax.axis_index("c")
    s = lax.axis_index("s")

    pltpu.sync_copy(col_hbm.at[c, s], col_v)
    pltpu.sync_copy(row_hbm.at[c, s], row_v)
    pltpu.sync_copy(val_hbm.at[c, s], val_v)

    def _zrow(i, carry):
        for cc in range(D // 16):
            gbuf[i, pl.ds(cc * 16, 16)] = jnp.zeros((16,), jnp.float32)
        return carry
    lax.fori_loop(0, ZROWS, _zrow, 0)
    my0 = s * (NACC // NS)
    for k in range(NACC // NS // ZROWS):
        pltpu.sync_copy(gbuf, acc.at[pl.ds(my0 + k * ZROWS, ZROWS)])

    plsc.subcore_barrier()

    def _block(j, carry):
        pltpu.sync_copy(x_hbm.at[col_v.at[j]], gbuf)

        def _sgrp(g, carry2):
            g16 = pl.multiple_of(g * 16, 16)
            vv = val_v[j, pl.ds(g16, 16)]
            for r16 in range(16):
                bb = jnp.broadcast_to(vv[r16], (16,))
                rr = g * 16 + r16
                for cc in range(D // 16):
                    sl = pl.ds(cc * 16, 16)
                    gbuf[rr, sl] = gbuf[rr, sl] * bb
            return carry2
        lax.fori_loop(0, BLK // 16, _sgrp, 0)

        pltpu.sync_copy(gbuf, acc.at[row_v.at[j]], add=True)
        return carry
    lax.fori_loop(0, JB, _block, 0)

    plsc.subcore_barrier()

    for k in range(NACC // NS // ZROWS):
        off = my0 + k * ZROWS
        pltpu.sync_copy(acc.at[pl.ds(off, ZROWS)], gbuf)
        pltpu.sync_copy(gbuf, part_hbm.at[c].at[pl.ds(off, ZROWS)])


def _combine_body(last, part_ref, mean_ref, x_ref, mout_ref):
    x = part_ref[0] + part_ref[1]
    x_ref[...] = x
    m = mean_ref[...] + x
    mout_ref[...] = m * 0.25 if last else m


def _tc_combine(part, mean, last):
    r = 1000
    grid = N_NODES // r
    return pl.pallas_call(
        functools.partial(_combine_body, last),
        grid=(grid,),
        in_specs=[
            pl.BlockSpec((NC, r, D), lambda i: (0, i, 0)),
            pl.BlockSpec((r, D), lambda i: (i, 0)),
        ],
        out_specs=[
            pl.BlockSpec((r, D), lambda i: (i, 0)),
            pl.BlockSpec((r, D), lambda i: (i, 0)),
        ],
        out_shape=[
            jax.ShapeDtypeStruct((N_NODES, D), jnp.float32),
            jax.ShapeDtypeStruct((N_NODES, D), jnp.float32),
        ],
    )(part, mean)


def kernel(emb, adj_val, adj_row, adj_col):
    col = adj_col.astype(jnp.int32)
    row = adj_row.astype(jnp.int32)
    val = adj_val.astype(jnp.float32)
    pad = NE_PAD - N_EDGES
    col = jnp.concatenate([col, jnp.zeros((pad,), jnp.int32)])
    row = jnp.concatenate([row, jnp.full((pad,), NACC - 1, jnp.int32)])
    val = jnp.concatenate([val, jnp.zeros((pad,), jnp.float32)])
    col3 = col.reshape(NC, NS, JB, BLK)
    row3 = row.reshape(NC, NS, JB, BLK)
    val3 = val.reshape(NC, NS, JB, BLK)

    x = emb
    mean = emb
    for layer in range(N_LAYERS):
        part = _sc_propagate(x, col3, row3, val3)
        x, mean = _tc_combine(part, mean, layer == N_LAYERS - 1)
    return mean[:N_USERS], mean[N_USERS:]


# speedup vs baseline: 2.7005x; 1.1408x over previous
import functools

import jax
import jax.numpy as jnp
from jax import lax
from jax.experimental import pallas as pl
from jax.experimental.pallas import tpu as pltpu
from jax.experimental.pallas import tpu_sc as plsc

N_USERS = 4000
N_NODES = 10000
N_EDGES = 320000
D = 128
N_LAYERS = 3

NC = 2
NS = 16
BLK = 128
JB = 80
JH = JB // 2
NE_PAD = NC * NS * JB * BLK
NACC = 10240
ZROWS = BLK

_mesh = plsc.VectorSubcoreMesh(
    core_axis_name="c", subcore_axis_name="s", num_cores=NC, num_subcores=NS)


@functools.partial(
    pl.kernel,
    out_type=jax.ShapeDtypeStruct((NC, NACC, D), jnp.float32),
    mesh=_mesh,
    scratch_types=[
        pltpu.VMEM((JH, BLK), jnp.int32),
        pltpu.VMEM((JH, BLK), jnp.int32),
        pltpu.VMEM((JH, BLK), jnp.float32),
        pltpu.VMEM((BLK, D), jnp.float32),
        pltpu.VMEM((BLK, D), jnp.float32),
        pltpu.VMEM_SHARED((NACC, D), jnp.float32),
        pltpu.SemaphoreType.DMA,
        pltpu.SemaphoreType.DMA,
        pltpu.SemaphoreType.DMA,
        pltpu.SemaphoreType.DMA,
    ],
)
def _sc_propagate(x_hbm, col_hbm, row_hbm, val_hbm, part_hbm,
                  col_v, row_v, val_v, buf0, buf1, acc, gs0, gs1, ss0, ss1):
    c = lax.axis_index("c")
    s = lax.axis_index("s")
    bufs = (buf0, buf1)
    gsems = (gs0, gs1)
    ssems = (ss0, ss1)

    def _zrow(i, carry):
        for cc in range(D // 16):
            buf0[i, pl.ds(cc * 16, 16)] = jnp.zeros((16,), jnp.float32)
        return carry
    lax.fori_loop(0, ZROWS, _zrow, 0)
    my0 = s * (NACC // NS)
    for k in range(NACC // NS // ZROWS):
        pltpu.sync_copy(buf0, acc.at[pl.ds(my0 + k * ZROWS, ZROWS)])

    plsc.subcore_barrier()

    def _scale(buf, j):
        def _sgrp(g, carry):
            g16 = pl.multiple_of(g * 16, 16)
            vv = val_v[j, pl.ds(g16, 16)]
            for r16 in range(16):
                bb = jnp.broadcast_to(vv[r16], (16,))
                rr = g16 + r16
                for cc in range(D // 16):
                    sl = pl.ds(cc * 16, 16)
                    buf[rr, sl] = buf[rr, sl] * bb
            return carry
        lax.fori_loop(0, BLK // 16, _sgrp, 0)

    for h in range(2):
        pltpu.sync_copy(col_hbm.at[c, s, pl.ds(h * JH, JH)], col_v)
        pltpu.sync_copy(row_hbm.at[c, s, pl.ds(h * JH, JH)], row_v)
        pltpu.sync_copy(val_hbm.at[c, s, pl.ds(h * JH, JH)], val_v)

        pltpu.async_copy(x_hbm.at[col_v.at[0]], buf0, gs0)
        pltpu.async_copy(x_hbm.at[col_v.at[1]], buf1, gs1)

        def _pair(i, carry):
            j0 = i * 2
            for p in range(2):
                j = j0 + p
                buf, gsem, ssem = bufs[p], gsems[p], ssems[p]
                pltpu.make_async_copy(x_hbm.at[pl.ds(0, BLK)], buf, gsem).wait()
                _scale(buf, j)
                pltpu.async_copy(buf, acc.at[row_v.at[j]], ssem, add=True)
            for p in range(2):
                j = j0 + p
                buf, gsem, ssem = bufs[p], gsems[p], ssems[p]
                pltpu.make_async_copy(buf, acc.at[pl.ds(0, BLK)], ssem).wait()

                @pl.when(j + 2 < JH)
                def _prefetch():
                    pltpu.async_copy(x_hbm.at[col_v.at[j + 2]], buf, gsem)
            return carry
        lax.fori_loop(0, JH // 2, _pair, 0)

    plsc.subcore_barrier()

    for k in range(NACC // NS // ZROWS):
        off = my0 + k * ZROWS
        pltpu.sync_copy(acc.at[pl.ds(off, ZROWS)], buf0)
        pltpu.sync_copy(buf0, part_hbm.at[c].at[pl.ds(off, ZROWS)])


def _combine_body(last, part_ref, mean_ref, x_ref, mout_ref):
    x = part_ref[0] + part_ref[1]
    x_ref[...] = x
    m = mean_ref[...] + x
    mout_ref[...] = m * 0.25 if last else m


def _tc_combine(part, mean, last):
    r = 1000
    grid = N_NODES // r
    return pl.pallas_call(
        functools.partial(_combine_body, last),
        grid=(grid,),
        in_specs=[
            pl.BlockSpec((NC, r, D), lambda i: (0, i, 0)),
            pl.BlockSpec((r, D), lambda i: (i, 0)),
        ],
        out_specs=[
            pl.BlockSpec((r, D), lambda i: (i, 0)),
            pl.BlockSpec((r, D), lambda i: (i, 0)),
        ],
        out_shape=[
            jax.ShapeDtypeStruct((N_NODES, D), jnp.float32),
            jax.ShapeDtypeStruct((N_NODES, D), jnp.float32),
        ],
    )(part, mean)


def kernel(emb, adj_val, adj_row, adj_col):
    col = adj_col.astype(jnp.int32)
    row = adj_row.astype(jnp.int32)
    val = adj_val.astype(jnp.float32)
    pad = NE_PAD - N_EDGES
    col = jnp.concatenate([col, jnp.zeros((pad,), jnp.int32)])
    row = jnp.concatenate([row, jnp.full((pad,), NACC - 1, jnp.int32)])
    val = jnp.concatenate([val, jnp.zeros((pad,), jnp.float32)])
    col3 = col.reshape(NC, NS, JB, BLK)
    row3 = row.reshape(NC, NS, JB, BLK)
    val3 = val.reshape(NC, NS, JB, BLK)

    x = emb
    mean = emb
    for layer in range(N_LAYERS):
        part = _sc_propagate(x, col3, row3, val3)
        x, mean = _tc_combine(part, mean, layer == N_LAYERS - 1)
    return mean[:N_USERS], mean[N_USERS:]
